# baseline (speedup 1.0000x reference)
import functools

import jax
import jax.numpy as jnp
from jax import lax
from jax.experimental import pallas as pl
from jax.experimental.pallas import tpu as pltpu
from jax.experimental.pallas import tpu_sc as plsc

_D = 16
_NC = 2
_NS = 16
_NW = _NC * _NS
_PK = 8
_TCH = 64
_GCH = 416


@jax.jit
def _split_layer(inputs, table):
    batch, cars = inputs.shape
    n = batch * cars
    out_w = cars * _D
    rows_per_w = batch // _NW
    n_per_w = n // _NW
    f_per_w = n_per_w * _D
    vocab = table.shape[0]
    packed = vocab // _PK
    per_w_pack = (packed // (8 * _NW)) * 8
    rem8 = (packed - per_w_pack * _NW) // 8
    nch = n_per_w // _GCH
    mesh = plsc.VectorSubcoreMesh(core_axis_name="c", subcore_axis_name="s")

    @functools.partial(
        pl.kernel,
        mesh=mesh,
        out_type=jax.ShapeDtypeStruct((n,), jnp.int32),
        scratch_types=[
            pltpu.VMEM((rows_per_w, cars), jnp.int32),
            pltpu.VMEM((n_per_w,), jnp.int32),
        ],
    )
    def flatten_kernel(idx_hbm, flat_hbm, idx_v, flat_v):
        wid = lax.axis_index("s") * _NC + lax.axis_index("c")
        pltpu.sync_copy(idx_hbm.at[pl.ds(wid * rows_per_w, rows_per_w)], idx_v)

        def body(r, carry):
            flat_v[pl.ds(r * cars, _D)] = idx_v[r, pl.ds(0, _D)]
            flat_v[pl.ds(r * cars + cars - _D, _D)] = (
                idx_v[r, pl.ds(cars - _D, _D)])
            return carry

        lax.fori_loop(0, rows_per_w, body, 0)
        pltpu.sync_copy(flat_v, flat_hbm.at[pl.ds(wid * n_per_w, n_per_w)])

    @functools.partial(
        pl.kernel,
        mesh=mesh,
        out_type=jax.ShapeDtypeStruct((packed, _PK * _D), jnp.float32),
        scratch_types=[
            pltpu.VMEM((_PK, _TCH, _D), jnp.float32),
            pltpu.VMEM((_TCH, _PK * _D), jnp.float32),
        ]
        + [pltpu.SemaphoreType.DMA for _ in range(4)],
    )
    def pack_kernel(t3_hbm, tl_hbm, big3, bufl, *sems):
        wid = lax.axis_index("s") * _NC + lax.axis_index("c")
        base = wid * per_w_pack

        def chunk(t0, rows):
            copies = [
                pltpu.async_copy(
                    t3_hbm.at[pl.ds(t0, rows), p],
                    big3.at[p, pl.ds(0, rows)], sems[p % 4])
                for p in range(_PK)
            ]
            for cp in copies:
                cp.wait()

            def comb(r, carry):
                for p in range(_PK):
                    bufl[r, pl.ds(p * _D, _D)] = big3[p, r, :]
                return carry

            lax.fori_loop(0, rows, comb, 0)
            pltpu.sync_copy(bufl.at[pl.ds(0, rows)],
                            tl_hbm.at[pl.ds(t0, rows)])

        def body(c, carry):
            chunk(base + c * _TCH, _TCH)
            return carry

        lax.fori_loop(0, per_w_pack // _TCH, body, 0)

        @pl.when(wid < rem8)
        def _():
            chunk(packed - rem8 * 8 + wid * 8, 8)

    @functools.partial(
        pl.kernel,
        mesh=mesh,
        compiler_params=pltpu.CompilerParams(
            use_tc_tiling_on_sc=False, needs_layout_passes=False),
        out_type=jax.ShapeDtypeStruct((n * _D,), jnp.float32),
        scratch_types=[
            pltpu.VMEM((n_per_w,), jnp.int32),
            pltpu.VMEM((_GCH,), jnp.int32),
            pltpu.VMEM((_GCH, _PK * _D), jnp.float32),
            pltpu.VMEM((_GCH * _D,), jnp.float32),
            pltpu.SemaphoreType.DMA,
        ],
    )
    def gather_kernel(flat_hbm, tl_hbm, out_hbm, idx_v, blk_v, wide_v, fl_v,
                      gsem):
        wid = lax.axis_index("s") * _NC + lax.axis_index("c")
        base = wid * n_per_w
        pltpu.sync_copy(flat_hbm.at[pl.ds(base, n_per_w)], idx_v)

        for c in range(nch):
            def mkblk(t, carry, c=c):
                blk_v[pl.ds(t * _D, _D)] = (
                    idx_v[pl.ds(c * _GCH + t * _D, _D)] >> 3)
                return carry

            lax.fori_loop(0, _GCH // _D, mkblk, 0)
            pltpu.async_copy(tl_hbm.at[blk_v], wide_v, gsem).wait()

            def sel(t, carry, c=c):
                lvec = t * _D + lax.iota(jnp.int32, _D)
                offv = (idx_v[pl.ds(c * _GCH + t * _D, _D)] & 7) * _D
                for d in range(_D):
                    vals = plsc.load_gather(wide_v, [lvec, offv + d])
                    plsc.store_scatter(fl_v, [lvec * _D + d], vals)
                return carry

            lax.fori_loop(0, _GCH // _D, sel, 0)
            pltpu.sync_copy(
                fl_v,
                out_hbm.at[pl.ds(wid * f_per_w + c * _GCH * _D, _GCH * _D)])

    @functools.partial(
        pl.kernel,
        mesh=mesh,
        out_type=jax.ShapeDtypeStruct((batch, 1, out_w), jnp.float32),
        scratch_types=[
            pltpu.VMEM((f_per_w,), jnp.float32),
            pltpu.SemaphoreType.DMA,
        ],
    )
    def format_kernel(vals_hbm, out_hbm, vals_v, sem):
        wid = lax.axis_index("s") * _NC + lax.axis_index("c")
        row0 = wid * rows_per_w
        pltpu.sync_copy(vals_hbm.at[pl.ds(wid * f_per_w, f_per_w)], vals_v)

        copies = []
        for t in range(rows_per_w):
            copies.append(pltpu.async_copy(
                vals_v.at[pl.ds(t * out_w, out_w)],
                out_hbm.at[row0 + t, 0], sem))
        for c in copies:
            c.wait()

    flat = flatten_kernel(inputs)
    table_l = pack_kernel(table.reshape(packed, _PK, _D))
    vals = gather_kernel(flat, table_l)
    return format_kernel(vals)


def kernel(inputs, table):
    return _split_layer(inputs, table)
